# baseline (speedup 1.0000x reference)
import functools

import jax
import jax.numpy as jnp
from jax import lax
from jax.experimental import pallas as pl
from jax.experimental.pallas import tpu as pltpu
from jax.experimental.pallas import tpu_sc as plsc

N = 10000
E = 160000
D = 128
COLS = 3 * D
HALF = COLS // 2
NC = 2
NS = 16
L = 16
NVREG = HALF // L

C = 16
SUP = 400
NCH = SUP // C
EPT = E // NS
SUPS = EPT // SUP
RPT = N // NS


def _mm_body(q_ref, w_ref, o_ref):
    o_ref[...] = lax.dot_general(
        q_ref[...], w_ref[...], (((1,), (1,)), ((), ())),
        preferred_element_type=jnp.float32).astype(jnp.bfloat16)


def _dense(q2, w):
    return pl.pallas_call(
        _mm_body,
        out_shape=jax.ShapeDtypeStruct((N, D), jnp.bfloat16),
    )(q2, w)


_PERM = []
for _b in range(D // 32):
    for _k in range(16):
        _PERM.extend([_b * 32 + _k, _b * 32 + 16 + _k])


@functools.partial(
    pl.kernel,
    out_type=jax.ShapeDtypeStruct((NC, N, HALF), jnp.float32),
    mesh=plsc.VectorSubcoreMesh(core_axis_name="c", subcore_axis_name="s"),
    compiler_params=pltpu.CompilerParams(use_tc_tiling_on_sc=False,
                                         needs_layout_passes=False),
    scratch_types=[
        pltpu.VMEM((SUP,), jnp.int32),
        pltpu.VMEM((NCH, C), jnp.int32),
        pltpu.VMEM((SUP,), jnp.float32),
        pltpu.VMEM((3, SUP), jnp.float32),
        pltpu.VMEM((C, D), jnp.bfloat16),
        pltpu.VMEM((C, D), jnp.bfloat16),
        pltpu.VMEM((C, D), jnp.bfloat16),
        pltpu.VMEM((C, D), jnp.bfloat16),
        pltpu.VMEM((C, HALF), jnp.float32),
        pltpu.VMEM_SHARED((N, HALF), jnp.float32),
        pltpu.SemaphoreType.DMA,
        pltpu.SemaphoreType.DMA,
        pltpu.SemaphoreType.DMA,
        pltpu.SemaphoreType.DMA,
        pltpu.SemaphoreType.DMA,
        pltpu.SemaphoreType.DMA,
    ],
)
def _sc_update(qi_hbm, mu_hbm, v_hbm, idxi_hbm, idxj_hbm, rcut_hbm, out_hbm,
               idxj_v, idxi_v, rcut_v, vs_v, rows_a, rows_b, rows_c, rows_d,
               buf_v, acc, sem_a, sem_b, sem_c, sem_d, sem_m, sem_s):
    c = lax.axis_index("c")
    s = lax.axis_index("s")

    pltpu.sync_copy(mu_hbm.at[c, pl.ds(s * RPT, RPT)],
                    acc.at[pl.ds(s * RPT, RPT)])
    plsc.subcore_barrier()

    zvec = jnp.zeros((L,), jnp.float32)
    for i in range(C):
        for j in range(NVREG):
            buf_v[i, j * L:(j + 1) * L] = zvec
    zidx = jnp.zeros((L,), jnp.int32)
    pltpu.async_copy(buf_v, acc.at[zidx], sem_s, add=True)

    def _emit_edges(cpy, t, rows_v):
        def body():
            for i in range(C):
                ev = jnp.full((L,), t * C + i, jnp.int32)
                need = sorted({(cpy * 6 + jp) // 4 for jp in range(6)})
                spl = {cc: plsc.load_gather(
                    vs_v, [jnp.full((L,), cc, jnp.int32), ev])
                    for cc in need}
                spb = {cc: plsc.pack(spl[cc], spl[cc],
                                     format=plsc.PackFormat.INTERLEAVED)
                       for cc in need}
                for jp in range(6):
                    gp = cpy * 6 + jp
                    c3 = gp // 4
                    dp = gp - c3 * 4
                    prod = (rows_v[i, dp * 32:(dp + 1) * 32] * spb[c3])
                    lo, hi = plsc.unpack(
                        prod, format=plsc.PackFormat.INTERLEAVED)
                    buf_v[i, (2 * jp) * L:(2 * jp + 1) * L] = lo
                    buf_v[i, (2 * jp + 1) * L:(2 * jp + 2) * L] = hi
        return body

    def _gather_start(e0, t, rows_v, sem):
        return pltpu.async_copy(
            qi_hbm.at[idxj_v.at[pl.ds(t * C, C)]], rows_v, sem)

    def _gather_wait(e0, t, rows_v, sem):
        pltpu.make_async_copy(
            qi_hbm.at[idxj_v.at[pl.ds(t * C, C)]], rows_v, sem).wait()

    def _chunk_compute(t, rows_v):
        pltpu.make_async_copy(buf_v, acc.at[zidx], sem_s).wait()
        pl.when(c == 0)(_emit_edges(0, t, rows_v))
        pl.when(c == 1)(_emit_edges(1, t, rows_v))
        pltpu.async_copy(buf_v, acc.at[idxi_v.at[t]], sem_s, add=True)

    def sup_body(u, carry):
        e0 = s * EPT + u * SUP
        m = []
        m.append((idxj_hbm.at[pl.ds(e0, SUP)], idxj_v))
        m.append((idxi_hbm.at[pl.ds(s * (EPT // C) + u * NCH, NCH)], idxi_v))
        m.append((rcut_hbm.at[pl.ds(e0, SUP)], rcut_v))
        for cc in range(3):
            m.append((v_hbm.at[cc, pl.ds(e0, SUP)], vs_v.at[cc]))
        for src, dst in m:
            pltpu.async_copy(src, dst, sem_m)
        for src, dst in m:
            pltpu.make_async_copy(src, dst, sem_m).wait()

        ring = [(rows_a, sem_a), (rows_b, sem_b),
                (rows_c, sem_c), (rows_d, sem_d)]
        for k in range(3):
            _gather_start(e0, k, ring[k][0], ring[k][1])

        def sgrp(g, carry2):
            r = rcut_v[pl.ds(g * L, L)]
            for cc in range(3):
                vs_v[cc, pl.ds(g * L, L)] = vs_v[cc, pl.ds(g * L, L)] * r
            return carry2
        lax.fori_loop(0, SUP // L, sgrp, 0, unroll=False)

        def quad_body(qq, carry2):
            t = 4 * qq
            for k in range(4):
                fr, fs = ring[(3 + k) % 4]
                fc = jnp.minimum(t + 3 + k, NCH - 1)
                _gather_start(e0, fc, fr, fs)
                _gather_wait(e0, t + k, ring[k][0], ring[k][1])
                _chunk_compute(t + k, ring[k][0])
            return carry2
        lax.fori_loop(0, (NCH - 1) // 4, quad_body, 0, unroll=False)
        _gather_wait(e0, NCH - 1, rows_a, sem_a)
        _chunk_compute(NCH - 1, rows_a)
        _gather_wait(e0, NCH - 1, rows_b, sem_b)
        _gather_wait(e0, NCH - 1, rows_c, sem_c)
        return carry
    lax.fori_loop(0, SUPS, sup_body, 0, unroll=False)
    pltpu.make_async_copy(buf_v, acc.at[zidx], sem_s).wait()

    plsc.subcore_barrier()
    pltpu.sync_copy(acc.at[pl.ds(s * RPT, RPT)],
                    out_hbm.at[c, pl.ds(s * RPT, RPT)])


def kernel(q, mu_electric_field, v_ij, idx_i, idx_j, rcut_ij, W_electric_field):
    q2 = q.reshape(N, D)
    qi = _dense(q2, W_electric_field[jnp.array(_PERM)])
    mu2 = mu_electric_field.reshape(N, COLS)
    mu_halves = jnp.stack([mu2[:, :HALF], mu2[:, HALF:]])
    vf = v_ij.T.astype(jnp.float32)
    ii = idx_i.astype(jnp.int32).reshape(E // C, C)
    jj = idx_j.astype(jnp.int32)
    out2 = _sc_update(qi, mu_halves, vf, ii, jj, rcut_ij)
    out = jnp.concatenate([out2[0], out2[1]], axis=1)
    return out.reshape(N, 3, D)


# speedup vs baseline: 1.2820x; 1.2820x over previous
import functools

import jax
import jax.numpy as jnp
from jax import lax
from jax.experimental import pallas as pl
from jax.experimental.pallas import tpu as pltpu
from jax.experimental.pallas import tpu_sc as plsc

N = 10000
E = 160000
D = 128
COLS = 3 * D
HALF = COLS // 2
NC = 2
NS = 16
L = 16
NVREG = HALF // L

C = 16
SUP = 400
NCH = SUP // C
EPT = E // NS
SUPS = EPT // SUP
RPT = N // NS


def _mm_body(q_ref, w_ref, o_ref):
    o_ref[...] = lax.dot_general(
        q_ref[...], w_ref[...], (((1,), (1,)), ((), ())),
        preferred_element_type=jnp.float32).astype(jnp.bfloat16)


def _dense(q2, w):
    return pl.pallas_call(
        _mm_body,
        out_shape=jax.ShapeDtypeStruct((N, D), jnp.bfloat16),
    )(q2, w)


_PERM = []
for _b in range(D // 32):
    for _k in range(16):
        _PERM.extend([_b * 32 + _k, _b * 32 + 16 + _k])


@functools.partial(
    pl.kernel,
    out_type=jax.ShapeDtypeStruct((NC, N, HALF), jnp.float32),
    mesh=plsc.VectorSubcoreMesh(core_axis_name="c", subcore_axis_name="s"),
    compiler_params=pltpu.CompilerParams(use_tc_tiling_on_sc=False,
                                         needs_layout_passes=False),
    scratch_types=[
        pltpu.VMEM((SUP,), jnp.int32),
        pltpu.VMEM((NCH, C), jnp.int32),
        pltpu.VMEM((SUP,), jnp.float32),
        pltpu.VMEM((3, SUP), jnp.float32),
        pltpu.VMEM((C, D), jnp.bfloat16),
        pltpu.VMEM((C, D), jnp.bfloat16),
        pltpu.VMEM((C, HALF), jnp.float32),
        pltpu.VMEM_SHARED((N, HALF), jnp.float32),
        pltpu.SemaphoreType.DMA,
        pltpu.SemaphoreType.DMA,
        pltpu.SemaphoreType.DMA,
        pltpu.SemaphoreType.DMA,
    ],
)
def _sc_update(qi_hbm, mu_hbm, v_hbm, idxi_hbm, idxj_hbm, rcut_hbm, out_hbm,
               idxj_v, idxi_v, rcut_v, vs_v, rows_a, rows_b, buf_v, acc,
               sem_a, sem_b, sem_m, sem_s):
    c = lax.axis_index("c")
    s = lax.axis_index("s")

    pltpu.sync_copy(mu_hbm.at[c, pl.ds(s * RPT, RPT)],
                    acc.at[pl.ds(s * RPT, RPT)])
    plsc.subcore_barrier()

    zvec = jnp.zeros((L,), jnp.float32)
    for i in range(C):
        for j in range(NVREG):
            buf_v[i, j * L:(j + 1) * L] = zvec
    zidx = jnp.zeros((L,), jnp.int32)
    pltpu.async_copy(buf_v, acc.at[zidx], sem_s, add=True)

    def _emit_edges(cpy, t, rows_v):
        def body():
            for i in range(C):
                ev = jnp.full((L,), t * C + i, jnp.int32)
                need = sorted({(cpy * 6 + jp) // 4 for jp in range(6)})
                spl = {cc: plsc.load_gather(
                    vs_v, [jnp.full((L,), cc, jnp.int32), ev])
                    for cc in need}
                spb = {cc: plsc.pack(spl[cc], spl[cc],
                                     format=plsc.PackFormat.INTERLEAVED)
                       for cc in need}
                for jp in range(6):
                    gp = cpy * 6 + jp
                    c3 = gp // 4
                    dp = gp - c3 * 4
                    prod = (rows_v[i, dp * 32:(dp + 1) * 32] * spb[c3])
                    lo, hi = plsc.unpack(
                        prod, format=plsc.PackFormat.INTERLEAVED)
                    buf_v[i, (2 * jp) * L:(2 * jp + 1) * L] = lo
                    buf_v[i, (2 * jp + 1) * L:(2 * jp + 2) * L] = hi
        return body

    def _gather_start(e0, t, rows_v, sem):
        return pltpu.async_copy(
            qi_hbm.at[idxj_v.at[pl.ds(t * C, C)]], rows_v, sem)

    def _gather_wait(e0, t, rows_v, sem):
        pltpu.make_async_copy(
            qi_hbm.at[idxj_v.at[pl.ds(t * C, C)]], rows_v, sem).wait()

    def _chunk_compute(t, rows_v):
        pltpu.make_async_copy(buf_v, acc.at[zidx], sem_s).wait()
        pl.when(c == 0)(_emit_edges(0, t, rows_v))
        pl.when(c == 1)(_emit_edges(1, t, rows_v))
        pltpu.async_copy(buf_v, acc.at[idxi_v.at[t]], sem_s, add=True)

    def sup_body(u, carry):
        e0 = s * EPT + u * SUP
        m = []
        m.append((idxj_hbm.at[pl.ds(e0, SUP)], idxj_v))
        m.append((idxi_hbm.at[pl.ds(s * (EPT // C) + u * NCH, NCH)], idxi_v))
        m.append((rcut_hbm.at[pl.ds(e0, SUP)], rcut_v))
        for cc in range(3):
            m.append((v_hbm.at[cc, pl.ds(e0, SUP)], vs_v.at[cc]))
        for src, dst in m:
            pltpu.async_copy(src, dst, sem_m)
        for src, dst in m:
            pltpu.make_async_copy(src, dst, sem_m).wait()

        _gather_start(e0, 0, rows_a, sem_a)

        def sgrp(g, carry2):
            r = rcut_v[pl.ds(g * L, L)]
            for cc in range(3):
                vs_v[cc, pl.ds(g * L, L)] = vs_v[cc, pl.ds(g * L, L)] * r
            return carry2
        lax.fori_loop(0, SUP // L, sgrp, 0, unroll=False)

        def pair_body(p, carry2):
            t = 2 * p
            _gather_start(e0, t + 1, rows_b, sem_b)
            _gather_wait(e0, t, rows_a, sem_a)
            _chunk_compute(t, rows_a)
            _gather_start(e0, t + 2, rows_a, sem_a)
            _gather_wait(e0, t + 1, rows_b, sem_b)
            _chunk_compute(t + 1, rows_b)
            return carry2
        lax.fori_loop(0, (NCH - 1) // 2, pair_body, 0, unroll=False)
        _gather_wait(e0, NCH - 1, rows_a, sem_a)
        _chunk_compute(NCH - 1, rows_a)
        return carry
    lax.fori_loop(0, SUPS, sup_body, 0, unroll=False)
    pltpu.make_async_copy(buf_v, acc.at[zidx], sem_s).wait()

    plsc.subcore_barrier()
    pltpu.sync_copy(acc.at[pl.ds(s * RPT, RPT)],
                    out_hbm.at[c, pl.ds(s * RPT, RPT)])


def kernel(q, mu_electric_field, v_ij, idx_i, idx_j, rcut_ij, W_electric_field):
    q2 = q.reshape(N, D)
    qi = _dense(q2, W_electric_field[jnp.array(_PERM)])
    mu2 = mu_electric_field.reshape(N, COLS)
    mu_halves = jnp.stack([mu2[:, :HALF], mu2[:, HALF:]])
    vf = v_ij.T.astype(jnp.float32)
    ii = idx_i.astype(jnp.int32).reshape(E // C, C)
    jj = idx_j.astype(jnp.int32)
    out2 = _sc_update(qi, mu_halves, vf, ii, jj, rcut_ij)
    out = jnp.concatenate([out2[0], out2[1]], axis=1)
    return out.reshape(N, 3, D)
